# Initial kernel scaffold; baseline (speedup 1.0000x reference)
import jax
import jax.numpy as jnp
from jax.experimental import pallas as pl


def kernel(X, graph, W1, b1, Wmu, bmu, Wsig, bsig, eps):
    raise NotImplementedError("write your pallas kernel here")


# speedup vs baseline: 14.5569x; 14.5569x over previous
import functools

import jax
import jax.numpy as jnp
from jax import lax
from jax.experimental import pallas as pl
from jax.experimental.pallas import tpu as pltpu
from jax.experimental.pallas import tpu_sc as plsc

N = 10000
E = 320000
NC, NS = 2, 16
ROWS = 10240
RPT = ROWS // NS
CH, CB = 80, 128
EPW = CH * CB
EPAD = NC * NS * EPW
D1 = 32
DH = 16
DZ = 64

_MESH = plsc.VectorSubcoreMesh(core_axis_name="c", subcore_axis_name="s")
_SC_PARAMS = pltpu.CompilerParams(use_tc_tiling_on_sc=False)


def _hist_body(dst_hbm, onerow_hbm, zrow_hbm, out_hbm, dstv, onesv, acc):
    c = lax.axis_index("c")
    s = lax.axis_index("s")
    pltpu.sync_copy(dst_hbm.at[c, s], dstv)
    pltpu.sync_copy(onerow_hbm, onesv)
    pltpu.sync_copy(zrow_hbm, acc.at[pl.ds(s * RPT, RPT)])
    plsc.subcore_barrier()

    @pl.loop(0, CH)
    def _(j):
        pltpu.sync_copy(onesv, acc.at[dstv.at[j]], add=True)

    plsc.subcore_barrier()
    pltpu.sync_copy(acc.at[pl.ds(s * RPT, RPT)],
                    out_hbm.at[c, pl.ds(s * RPT, RPT)])


@functools.partial(
    pl.kernel,
    out_type=jax.ShapeDtypeStruct((NC, ROWS, DH), jnp.float32),
    mesh=_MESH,
    compiler_params=_SC_PARAMS,
    scratch_types=[
        pltpu.VMEM((CH, CB), jnp.int32),
        pltpu.VMEM((CB, DH), jnp.float32),
        pltpu.VMEM_SHARED((ROWS, DH), jnp.float32),
    ],
)
def _sc_hist(dst_hbm, onerow_hbm, zrow_hbm, out_hbm, dstv, onesv, acc):
    _hist_body(dst_hbm, onerow_hbm, zrow_hbm, out_hbm, dstv, onesv, acc)


def _agg_body(src_hbm, dst_hbm, y_hbm, zrow_hbm, out_hbm, srcv, dstv, rowsv,
              acc):
    c = lax.axis_index("c")
    s = lax.axis_index("s")
    pltpu.sync_copy(src_hbm.at[c, s], srcv)
    pltpu.sync_copy(dst_hbm.at[c, s], dstv)
    pltpu.sync_copy(zrow_hbm, acc.at[pl.ds(s * RPT, RPT)])
    plsc.subcore_barrier()

    @pl.loop(0, CH)
    def _(j):
        pltpu.sync_copy(y_hbm.at[srcv.at[j]], rowsv)
        pltpu.sync_copy(rowsv, acc.at[dstv.at[j]], add=True)

    plsc.subcore_barrier()
    pltpu.sync_copy(acc.at[pl.ds(s * RPT, RPT)],
                    out_hbm.at[c, pl.ds(s * RPT, RPT)])


@functools.partial(
    pl.kernel,
    out_type=jax.ShapeDtypeStruct((NC, ROWS, D1), jnp.float32),
    mesh=_MESH,
    compiler_params=_SC_PARAMS,
    scratch_types=[
        pltpu.VMEM((CH, CB), jnp.int32),
        pltpu.VMEM((CH, CB), jnp.int32),
        pltpu.VMEM((CB, D1), jnp.float32),
        pltpu.VMEM_SHARED((ROWS, D1), jnp.float32),
    ],
)
def _sc_agg(src_hbm, dst_hbm, y_hbm, zrow_hbm, out_hbm, srcv, dstv, rowsv,
            acc):
    _agg_body(src_hbm, dst_hbm, y_hbm, zrow_hbm, out_hbm, srcv, dstv, rowsv,
              acc)


BLK = 1280


def _k2_body(x_ref, w1_ref, hist_ref, y1_ref, dinv_ref):
    i = pl.program_id(0)
    u = jnp.dot(x_ref[...], w1_ref[...],
                preferred_element_type=jnp.float32,
                precision=lax.Precision.HIGHEST)
    deg = hist_ref[0, :, 0:1] + hist_ref[1, :, 0:1] + 1.0
    rows = i * BLK + lax.broadcasted_iota(jnp.int32, (BLK, 1), 0)
    dinv = jnp.where(rows < N, 1.0 / jnp.sqrt(deg), 0.0)
    dinv32 = jnp.broadcast_to(dinv, (BLK, D1))
    y1_ref[...] = dinv32 * u
    dinv_ref[...] = dinv32


def _k2(X, W1, hist):
    return pl.pallas_call(
        _k2_body,
        grid=(ROWS // BLK,),
        in_specs=[
            pl.BlockSpec((BLK, 128), lambda i: (i, 0)),
            pl.BlockSpec((128, D1), lambda i: (0, 0)),
            pl.BlockSpec((NC, BLK, DH), lambda i: (0, i, 0)),
        ],
        out_specs=[
            pl.BlockSpec((BLK, D1), lambda i: (i, 0)),
            pl.BlockSpec((BLK, D1), lambda i: (i, 0)),
        ],
        out_shape=[
            jax.ShapeDtypeStruct((ROWS, D1), jnp.float32),
            jax.ShapeDtypeStruct((ROWS, D1), jnp.float32),
        ],
    )(X, W1, hist)


def _k3_body(s1_ref, y1_ref, dinv_ref, b1_ref, y2_ref):
    dinv = dinv_ref[...]
    agg = dinv * (s1_ref[0] + s1_ref[1] + y1_ref[...]) + b1_ref[...]
    y2_ref[...] = dinv * jnp.maximum(agg, 0.0)


def _k3(s1, y1, dinv32, b1):
    return pl.pallas_call(
        _k3_body,
        grid=(ROWS // BLK,),
        in_specs=[
            pl.BlockSpec((NC, BLK, D1), lambda i: (0, i, 0)),
            pl.BlockSpec((BLK, D1), lambda i: (i, 0)),
            pl.BlockSpec((BLK, D1), lambda i: (i, 0)),
            pl.BlockSpec((1, D1), lambda i: (0, 0)),
        ],
        out_specs=pl.BlockSpec((BLK, D1), lambda i: (i, 0)),
        out_shape=jax.ShapeDtypeStruct((ROWS, D1), jnp.float32),
    )(s1, y1, dinv32, b1)


def _k4_body(s2_ref, y2_ref, dinv_ref, wmu_ref, bmu_ref, wsig_ref, bsig_ref,
             eps_ref, mus_ref, lsig_ref, z_ref):
    i = pl.program_id(0)
    agg = dinv_ref[...] * (s2_ref[0] + s2_ref[1] + y2_ref[...])
    mus = jnp.dot(agg, wmu_ref[...], preferred_element_type=jnp.float32,
                  precision=lax.Precision.HIGHEST) + bmu_ref[...]
    lsig = jnp.dot(agg, wsig_ref[...], preferred_element_type=jnp.float32,
                   precision=lax.Precision.HIGHEST) + bsig_ref[...]
    z = mus + jnp.exp(0.5 * lsig) * eps_ref[...]
    rows = i * BLK + lax.broadcasted_iota(jnp.int32, (BLK, 1), 0)
    mus_ref[...] = mus
    lsig_ref[...] = lsig
    z_ref[...] = jnp.where(rows < N, z, 0.0)


def _k4(s2, y2, dinv32, Wmu, bmu, Wsig, bsig, eps):
    return pl.pallas_call(
        _k4_body,
        grid=(ROWS // BLK,),
        in_specs=[
            pl.BlockSpec((NC, BLK, D1), lambda i: (0, i, 0)),
            pl.BlockSpec((BLK, D1), lambda i: (i, 0)),
            pl.BlockSpec((BLK, D1), lambda i: (i, 0)),
            pl.BlockSpec((D1, DZ), lambda i: (0, 0)),
            pl.BlockSpec((1, DZ), lambda i: (0, 0)),
            pl.BlockSpec((D1, DZ), lambda i: (0, 0)),
            pl.BlockSpec((1, DZ), lambda i: (0, 0)),
            pl.BlockSpec((BLK, DZ), lambda i: (i, 0)),
        ],
        out_specs=[
            pl.BlockSpec((BLK, DZ), lambda i: (i, 0)),
            pl.BlockSpec((BLK, DZ), lambda i: (i, 0)),
            pl.BlockSpec((BLK, DZ), lambda i: (i, 0)),
        ],
        out_shape=[
            jax.ShapeDtypeStruct((N, DZ), jnp.float32),
            jax.ShapeDtypeStruct((N, DZ), jnp.float32),
            jax.ShapeDtypeStruct((ROWS, DZ), jnp.float32),
        ],
    )(s2, y2, dinv32, Wmu, bmu, Wsig, bsig, eps)


BM, BN = 1024, 2048


def _k5_body(zl_ref, zr_ref, out_ref):
    out_ref[...] = lax.dot_general(
        zl_ref[...], zr_ref[...], (((1,), (1,)), ((), ())),
        preferred_element_type=jnp.float32,
        precision=lax.Precision.HIGHEST)


def _k5(z):
    return pl.pallas_call(
        _k5_body,
        grid=(pl.cdiv(N, BM), pl.cdiv(N, BN)),
        in_specs=[
            pl.BlockSpec((BM, DZ), lambda i, j: (i, 0)),
            pl.BlockSpec((BN, DZ), lambda i, j: (j, 0)),
        ],
        out_specs=pl.BlockSpec((BM, BN), lambda i, j: (i, j)),
        out_shape=jax.ShapeDtypeStruct((N, N), jnp.float32),
    )(z, z)


def kernel(X, graph, W1, b1, Wmu, bmu, Wsig, bsig, eps):
    pad = jnp.full((EPAD - E,), N, dtype=jnp.int32)
    src = jnp.concatenate([graph[0], pad]).reshape(NC, NS, CH, CB)
    dst = jnp.concatenate([graph[1], pad]).reshape(NC, NS, CH, CB)
    zrow16 = jnp.zeros((RPT, DH), jnp.float32)
    zrow32 = jnp.zeros((RPT, D1), jnp.float32)
    onerow = jnp.concatenate(
        [jnp.ones((CB, 1), jnp.float32), jnp.zeros((CB, DH - 1), jnp.float32)],
        axis=1)

    hist = _sc_hist(dst, onerow, zrow16)
    y1, dinv32 = _k2(X, W1, hist)
    s1 = _sc_agg(src, dst, y1, zrow32)
    y2 = _k3(s1, y1, dinv32, b1.reshape(1, D1))
    s2 = _sc_agg(src, dst, y2, zrow32)
    mus, lsig, z = _k4(s2, y2, dinv32, Wmu, bmu.reshape(1, DZ),
                       Wsig, bsig.reshape(1, DZ), eps)
    zzt = _k5(z)
    return (zzt, mus, lsig)


# speedup vs baseline: 16.9814x; 1.1666x over previous
import functools

import jax
import jax.numpy as jnp
from jax import lax
from jax.experimental import pallas as pl
from jax.experimental.pallas import tpu as pltpu
from jax.experimental.pallas import tpu_sc as plsc

N = 10000
E = 320000
NC, NS = 2, 16
ROWS = 10240
RPT = ROWS // NS
CH, CB = 80, 128
EPW = CH * CB
EPAD = NC * NS * EPW
D1 = 32
DH = 16
DZ = 64

_MESH = plsc.VectorSubcoreMesh(core_axis_name="c", subcore_axis_name="s")
_SC_PARAMS = pltpu.CompilerParams(use_tc_tiling_on_sc=False)


def _hist_body(dst_hbm, onerow_hbm, zrow_hbm, out_hbm, dstv, onesv, acc):
    c = lax.axis_index("c")
    s = lax.axis_index("s")
    pltpu.sync_copy(dst_hbm.at[c, s], dstv)
    pltpu.sync_copy(onerow_hbm, onesv)
    pltpu.sync_copy(zrow_hbm, acc.at[pl.ds(s * RPT, RPT)])
    plsc.subcore_barrier()

    @pl.loop(0, CH)
    def _(j):
        pltpu.sync_copy(onesv, acc.at[dstv.at[j]], add=True)

    plsc.subcore_barrier()
    pltpu.sync_copy(acc.at[pl.ds(s * RPT, RPT)],
                    out_hbm.at[c, pl.ds(s * RPT, RPT)])


@functools.partial(
    pl.kernel,
    out_type=jax.ShapeDtypeStruct((NC, ROWS, DH), jnp.float32),
    mesh=_MESH,
    compiler_params=_SC_PARAMS,
    scratch_types=[
        pltpu.VMEM((CH, CB), jnp.int32),
        pltpu.VMEM((CB, DH), jnp.float32),
        pltpu.VMEM_SHARED((ROWS, DH), jnp.float32),
    ],
)
def _sc_hist(dst_hbm, onerow_hbm, zrow_hbm, out_hbm, dstv, onesv, acc):
    _hist_body(dst_hbm, onerow_hbm, zrow_hbm, out_hbm, dstv, onesv, acc)


def _agg_body(src_hbm, dst_hbm, y_hbm, zrow_hbm, out_hbm, srcv, dstv, rows0,
              rows1, acc, sem0, sem1):
    c = lax.axis_index("c")
    s = lax.axis_index("s")
    pltpu.sync_copy(src_hbm.at[c, s], srcv)
    pltpu.sync_copy(dst_hbm.at[c, s], dstv)
    pltpu.sync_copy(zrow_hbm, acc.at[pl.ds(s * RPT, RPT)])
    plsc.subcore_barrier()

    pltpu.async_copy(y_hbm.at[srcv.at[0]], rows0, sem0)

    @pl.loop(0, CH, step=2)
    def _(j):
        pltpu.async_copy(y_hbm.at[srcv.at[j + 1]], rows1, sem1)
        pltpu.make_async_copy(y_hbm.at[srcv.at[0]], rows0, sem0).wait()
        pltpu.sync_copy(rows0, acc.at[dstv.at[j]], add=True)

        @pl.when(j + 2 < CH)
        def _():
            pltpu.async_copy(y_hbm.at[srcv.at[j + 2]], rows0, sem0)

        pltpu.make_async_copy(y_hbm.at[srcv.at[0]], rows1, sem1).wait()
        pltpu.sync_copy(rows1, acc.at[dstv.at[j + 1]], add=True)

    plsc.subcore_barrier()
    pltpu.sync_copy(acc.at[pl.ds(s * RPT, RPT)],
                    out_hbm.at[c, pl.ds(s * RPT, RPT)])


@functools.partial(
    pl.kernel,
    out_type=jax.ShapeDtypeStruct((NC, ROWS, D1), jnp.float32),
    mesh=_MESH,
    compiler_params=_SC_PARAMS,
    scratch_types=[
        pltpu.VMEM((CH, CB), jnp.int32),
        pltpu.VMEM((CH, CB), jnp.int32),
        pltpu.VMEM((CB, D1), jnp.float32),
        pltpu.VMEM((CB, D1), jnp.float32),
        pltpu.VMEM_SHARED((ROWS, D1), jnp.float32),
        pltpu.SemaphoreType.DMA,
        pltpu.SemaphoreType.DMA,
    ],
)
def _sc_agg(src_hbm, dst_hbm, y_hbm, zrow_hbm, out_hbm, srcv, dstv, rows0,
            rows1, acc, sem0, sem1):
    _agg_body(src_hbm, dst_hbm, y_hbm, zrow_hbm, out_hbm, srcv, dstv, rows0,
              rows1, acc, sem0, sem1)


BLK = 1280


def _k2_body(x_ref, w1_ref, hist_ref, y1_ref, dinv_ref):
    i = pl.program_id(0)
    u = jnp.dot(x_ref[...], w1_ref[...],
                preferred_element_type=jnp.float32,
                precision=lax.Precision.HIGHEST)
    deg = hist_ref[0, :, 0:1] + hist_ref[1, :, 0:1] + 1.0
    rows = i * BLK + lax.broadcasted_iota(jnp.int32, (BLK, 1), 0)
    dinv = jnp.where(rows < N, 1.0 / jnp.sqrt(deg), 0.0)
    dinv32 = jnp.broadcast_to(dinv, (BLK, D1))
    y1_ref[...] = dinv32 * u
    dinv_ref[...] = dinv32


def _k2(X, W1, hist):
    return pl.pallas_call(
        _k2_body,
        grid=(ROWS // BLK,),
        in_specs=[
            pl.BlockSpec((BLK, 128), lambda i: (i, 0)),
            pl.BlockSpec((128, D1), lambda i: (0, 0)),
            pl.BlockSpec((NC, BLK, DH), lambda i: (0, i, 0)),
        ],
        out_specs=[
            pl.BlockSpec((BLK, D1), lambda i: (i, 0)),
            pl.BlockSpec((BLK, D1), lambda i: (i, 0)),
        ],
        out_shape=[
            jax.ShapeDtypeStruct((ROWS, D1), jnp.float32),
            jax.ShapeDtypeStruct((ROWS, D1), jnp.float32),
        ],
    )(X, W1, hist)


def _k3_body(s1_ref, y1_ref, dinv_ref, b1_ref, y2_ref):
    dinv = dinv_ref[...]
    agg = dinv * (s1_ref[0] + s1_ref[1] + y1_ref[...]) + b1_ref[...]
    y2_ref[...] = dinv * jnp.maximum(agg, 0.0)


def _k3(s1, y1, dinv32, b1):
    return pl.pallas_call(
        _k3_body,
        grid=(ROWS // BLK,),
        in_specs=[
            pl.BlockSpec((NC, BLK, D1), lambda i: (0, i, 0)),
            pl.BlockSpec((BLK, D1), lambda i: (i, 0)),
            pl.BlockSpec((BLK, D1), lambda i: (i, 0)),
            pl.BlockSpec((1, D1), lambda i: (0, 0)),
        ],
        out_specs=pl.BlockSpec((BLK, D1), lambda i: (i, 0)),
        out_shape=jax.ShapeDtypeStruct((ROWS, D1), jnp.float32),
    )(s1, y1, dinv32, b1)


def _k4_body(s2_ref, y2_ref, dinv_ref, wmu_ref, bmu_ref, wsig_ref, bsig_ref,
             eps_ref, mus_ref, lsig_ref, z_ref):
    i = pl.program_id(0)
    agg = dinv_ref[...] * (s2_ref[0] + s2_ref[1] + y2_ref[...])
    mus = jnp.dot(agg, wmu_ref[...], preferred_element_type=jnp.float32,
                  precision=lax.Precision.HIGHEST) + bmu_ref[...]
    lsig = jnp.dot(agg, wsig_ref[...], preferred_element_type=jnp.float32,
                   precision=lax.Precision.HIGHEST) + bsig_ref[...]
    z = mus + jnp.exp(0.5 * lsig) * eps_ref[...]
    rows = i * BLK + lax.broadcasted_iota(jnp.int32, (BLK, 1), 0)
    mus_ref[...] = mus
    lsig_ref[...] = lsig
    z_ref[...] = jnp.where(rows < N, z, 0.0)


def _k4(s2, y2, dinv32, Wmu, bmu, Wsig, bsig, eps):
    return pl.pallas_call(
        _k4_body,
        grid=(ROWS // BLK,),
        in_specs=[
            pl.BlockSpec((NC, BLK, D1), lambda i: (0, i, 0)),
            pl.BlockSpec((BLK, D1), lambda i: (i, 0)),
            pl.BlockSpec((BLK, D1), lambda i: (i, 0)),
            pl.BlockSpec((D1, DZ), lambda i: (0, 0)),
            pl.BlockSpec((1, DZ), lambda i: (0, 0)),
            pl.BlockSpec((D1, DZ), lambda i: (0, 0)),
            pl.BlockSpec((1, DZ), lambda i: (0, 0)),
            pl.BlockSpec((BLK, DZ), lambda i: (i, 0)),
        ],
        out_specs=[
            pl.BlockSpec((BLK, DZ), lambda i: (i, 0)),
            pl.BlockSpec((BLK, DZ), lambda i: (i, 0)),
            pl.BlockSpec((BLK, DZ), lambda i: (i, 0)),
        ],
        out_shape=[
            jax.ShapeDtypeStruct((N, DZ), jnp.float32),
            jax.ShapeDtypeStruct((N, DZ), jnp.float32),
            jax.ShapeDtypeStruct((ROWS, DZ), jnp.float32),
        ],
    )(s2, y2, dinv32, Wmu, bmu, Wsig, bsig, eps)


BM, BN = 1024, 2048


def _k5_body(zl_ref, zr_ref, out_ref):
    out_ref[...] = lax.dot_general(
        zl_ref[...], zr_ref[...], (((1,), (1,)), ((), ())),
        preferred_element_type=jnp.float32,
        precision=lax.Precision.HIGHEST)


def _k5(z):
    return pl.pallas_call(
        _k5_body,
        grid=(pl.cdiv(N, BM), pl.cdiv(N, BN)),
        in_specs=[
            pl.BlockSpec((BM, DZ), lambda i, j: (i, 0)),
            pl.BlockSpec((BN, DZ), lambda i, j: (j, 0)),
        ],
        out_specs=pl.BlockSpec((BM, BN), lambda i, j: (i, j)),
        out_shape=jax.ShapeDtypeStruct((N, N), jnp.float32),
    )(z, z)


def kernel(X, graph, W1, b1, Wmu, bmu, Wsig, bsig, eps):
    pad = jnp.full((EPAD - E,), N, dtype=jnp.int32)
    src = jnp.concatenate([graph[0], pad]).reshape(NC, NS, CH, CB)
    dst = jnp.concatenate([graph[1], pad]).reshape(NC, NS, CH, CB)
    zrow16 = jnp.zeros((RPT, DH), jnp.float32)
    zrow32 = jnp.zeros((RPT, D1), jnp.float32)
    onerow = jnp.concatenate(
        [jnp.ones((CB, 1), jnp.float32), jnp.zeros((CB, DH - 1), jnp.float32)],
        axis=1)

    hist = _sc_hist(dst, onerow, zrow16)
    y1, dinv32 = _k2(X, W1, hist)
    s1 = _sc_agg(src, dst, y1, zrow32)
    y2 = _k3(s1, y1, dinv32, b1.reshape(1, D1))
    s2 = _sc_agg(src, dst, y2, zrow32)
    mus, lsig, z = _k4(s2, y2, dinv32, Wmu, bmu.reshape(1, DZ),
                       Wsig, bsig.reshape(1, DZ), eps)
    zzt = _k5(z)
    return (zzt, mus, lsig)


# speedup vs baseline: 21.5624x; 1.2698x over previous
import functools

import jax
import jax.numpy as jnp
from jax import lax
from jax.experimental import pallas as pl
from jax.experimental.pallas import tpu as pltpu
from jax.experimental.pallas import tpu_sc as plsc

N = 10000
E = 320000
NC, NS = 2, 16
ROWS = 10240
RPT = ROWS // NS
CH, CB = 80, 128
EPW = CH * CB
EPAD = NC * NS * EPW
D1 = 32
DH = 16
DZ = 64

_MESH = plsc.VectorSubcoreMesh(core_axis_name="c", subcore_axis_name="s")
_SC_PARAMS = pltpu.CompilerParams(use_tc_tiling_on_sc=False)


def _hist_body(dst_hbm, onerow_hbm, zrow_hbm, out_hbm, dstv, onesv, acc):
    c = lax.axis_index("c")
    s = lax.axis_index("s")
    pltpu.sync_copy(dst_hbm.at[c, s], dstv)
    pltpu.sync_copy(onerow_hbm, onesv)
    pltpu.sync_copy(zrow_hbm, acc.at[pl.ds(s * RPT, RPT)])
    plsc.subcore_barrier()

    @pl.loop(0, CH)
    def _(j):
        pltpu.sync_copy(onesv, acc.at[dstv.at[j]], add=True)

    plsc.subcore_barrier()
    pltpu.sync_copy(acc.at[pl.ds(s * RPT, RPT)],
                    out_hbm.at[c, pl.ds(s * RPT, RPT)])


@functools.partial(
    pl.kernel,
    out_type=jax.ShapeDtypeStruct((NC, ROWS, DH), jnp.float32),
    mesh=_MESH,
    compiler_params=_SC_PARAMS,
    scratch_types=[
        pltpu.VMEM((CH, CB), jnp.int32),
        pltpu.VMEM((CB, DH), jnp.float32),
        pltpu.VMEM_SHARED((ROWS, DH), jnp.float32),
    ],
)
def _sc_hist(dst_hbm, onerow_hbm, zrow_hbm, out_hbm, dstv, onesv, acc):
    _hist_body(dst_hbm, onerow_hbm, zrow_hbm, out_hbm, dstv, onesv, acc)


def _agg_body(src_hbm, dst_hbm, y_hbm, zrow_hbm, out_hbm, srcv, dstv, rows0,
              rows1, ysh, acc, sem0, sem1):
    c = lax.axis_index("c")
    s = lax.axis_index("s")
    pltpu.sync_copy(src_hbm.at[c, s], srcv)
    pltpu.sync_copy(dst_hbm.at[c, s], dstv)
    pltpu.sync_copy(zrow_hbm, acc.at[pl.ds(s * RPT, RPT)])
    pltpu.sync_copy(y_hbm.at[pl.ds(s * RPT, RPT)], ysh.at[pl.ds(s * RPT, RPT)])
    plsc.subcore_barrier()

    pltpu.async_copy(ysh.at[srcv.at[0]], rows0, sem0)

    @pl.loop(0, CH, step=2)
    def _(j):
        pltpu.async_copy(ysh.at[srcv.at[j + 1]], rows1, sem1)
        pltpu.make_async_copy(y_hbm.at[pl.ds(0, CB)], rows0, sem0).wait()
        pltpu.sync_copy(rows0, acc.at[dstv.at[j]], add=True)

        @pl.when(j + 2 < CH)
        def _():
            pltpu.async_copy(ysh.at[srcv.at[j + 2]], rows0, sem0)

        pltpu.make_async_copy(y_hbm.at[pl.ds(0, CB)], rows1, sem1).wait()
        pltpu.sync_copy(rows1, acc.at[dstv.at[j + 1]], add=True)

    plsc.subcore_barrier()
    pltpu.sync_copy(acc.at[pl.ds(s * RPT, RPT)],
                    out_hbm.at[c, pl.ds(s * RPT, RPT)])


@functools.partial(
    pl.kernel,
    out_type=jax.ShapeDtypeStruct((NC, ROWS, D1), jnp.float32),
    mesh=_MESH,
    compiler_params=_SC_PARAMS,
    scratch_types=[
        pltpu.VMEM((CH, CB), jnp.int32),
        pltpu.VMEM((CH, CB), jnp.int32),
        pltpu.VMEM((CB, D1), jnp.float32),
        pltpu.VMEM((CB, D1), jnp.float32),
        pltpu.VMEM_SHARED((ROWS, D1), jnp.float32),
        pltpu.VMEM_SHARED((ROWS, D1), jnp.float32),
        pltpu.SemaphoreType.DMA,
        pltpu.SemaphoreType.DMA,
    ],
)
def _sc_agg(src_hbm, dst_hbm, y_hbm, zrow_hbm, out_hbm, srcv, dstv, rows0,
            rows1, ysh, acc, sem0, sem1):
    _agg_body(src_hbm, dst_hbm, y_hbm, zrow_hbm, out_hbm, srcv, dstv, rows0,
              rows1, ysh, acc, sem0, sem1)


BLK = 1280


def _k2_body(x_ref, w1_ref, hist_ref, y1_ref, dinv_ref):
    i = pl.program_id(0)
    u = jnp.dot(x_ref[...], w1_ref[...],
                preferred_element_type=jnp.float32,
                precision=lax.Precision.HIGHEST)
    deg = hist_ref[0, :, 0:1] + hist_ref[1, :, 0:1] + 1.0
    rows = i * BLK + lax.broadcasted_iota(jnp.int32, (BLK, 1), 0)
    dinv = jnp.where(rows < N, 1.0 / jnp.sqrt(deg), 0.0)
    dinv32 = jnp.broadcast_to(dinv, (BLK, D1))
    y1_ref[...] = dinv32 * u
    dinv_ref[...] = dinv32


def _k2(X, W1, hist):
    return pl.pallas_call(
        _k2_body,
        grid=(ROWS // BLK,),
        in_specs=[
            pl.BlockSpec((BLK, 128), lambda i: (i, 0)),
            pl.BlockSpec((128, D1), lambda i: (0, 0)),
            pl.BlockSpec((NC, BLK, DH), lambda i: (0, i, 0)),
        ],
        out_specs=[
            pl.BlockSpec((BLK, D1), lambda i: (i, 0)),
            pl.BlockSpec((BLK, D1), lambda i: (i, 0)),
        ],
        out_shape=[
            jax.ShapeDtypeStruct((ROWS, D1), jnp.float32),
            jax.ShapeDtypeStruct((ROWS, D1), jnp.float32),
        ],
    )(X, W1, hist)


def _k3_body(s1_ref, y1_ref, dinv_ref, b1_ref, y2_ref):
    dinv = dinv_ref[...]
    agg = dinv * (s1_ref[0] + s1_ref[1] + y1_ref[...]) + b1_ref[...]
    y2_ref[...] = dinv * jnp.maximum(agg, 0.0)


def _k3(s1, y1, dinv32, b1):
    return pl.pallas_call(
        _k3_body,
        grid=(ROWS // BLK,),
        in_specs=[
            pl.BlockSpec((NC, BLK, D1), lambda i: (0, i, 0)),
            pl.BlockSpec((BLK, D1), lambda i: (i, 0)),
            pl.BlockSpec((BLK, D1), lambda i: (i, 0)),
            pl.BlockSpec((1, D1), lambda i: (0, 0)),
        ],
        out_specs=pl.BlockSpec((BLK, D1), lambda i: (i, 0)),
        out_shape=jax.ShapeDtypeStruct((ROWS, D1), jnp.float32),
    )(s1, y1, dinv32, b1)


def _k4_body(s2_ref, y2_ref, dinv_ref, wmu_ref, bmu_ref, wsig_ref, bsig_ref,
             eps_ref, mus_ref, lsig_ref, z_ref):
    i = pl.program_id(0)
    agg = dinv_ref[...] * (s2_ref[0] + s2_ref[1] + y2_ref[...])
    mus = jnp.dot(agg, wmu_ref[...], preferred_element_type=jnp.float32,
                  precision=lax.Precision.HIGHEST) + bmu_ref[...]
    lsig = jnp.dot(agg, wsig_ref[...], preferred_element_type=jnp.float32,
                   precision=lax.Precision.HIGHEST) + bsig_ref[...]
    z = mus + jnp.exp(0.5 * lsig) * eps_ref[...]
    rows = i * BLK + lax.broadcasted_iota(jnp.int32, (BLK, 1), 0)
    mus_ref[...] = mus
    lsig_ref[...] = lsig
    z_ref[...] = jnp.where(rows < N, z, 0.0)


def _k4(s2, y2, dinv32, Wmu, bmu, Wsig, bsig, eps):
    return pl.pallas_call(
        _k4_body,
        grid=(ROWS // BLK,),
        in_specs=[
            pl.BlockSpec((NC, BLK, D1), lambda i: (0, i, 0)),
            pl.BlockSpec((BLK, D1), lambda i: (i, 0)),
            pl.BlockSpec((BLK, D1), lambda i: (i, 0)),
            pl.BlockSpec((D1, DZ), lambda i: (0, 0)),
            pl.BlockSpec((1, DZ), lambda i: (0, 0)),
            pl.BlockSpec((D1, DZ), lambda i: (0, 0)),
            pl.BlockSpec((1, DZ), lambda i: (0, 0)),
            pl.BlockSpec((BLK, DZ), lambda i: (i, 0)),
        ],
        out_specs=[
            pl.BlockSpec((BLK, DZ), lambda i: (i, 0)),
            pl.BlockSpec((BLK, DZ), lambda i: (i, 0)),
            pl.BlockSpec((BLK, DZ), lambda i: (i, 0)),
        ],
        out_shape=[
            jax.ShapeDtypeStruct((N, DZ), jnp.float32),
            jax.ShapeDtypeStruct((N, DZ), jnp.float32),
            jax.ShapeDtypeStruct((ROWS, DZ), jnp.float32),
        ],
    )(s2, y2, dinv32, Wmu, bmu, Wsig, bsig, eps)


BM, BN = 1024, 2048


def _k5_body(zl_ref, zr_ref, out_ref):
    out_ref[...] = lax.dot_general(
        zl_ref[...], zr_ref[...], (((1,), (1,)), ((), ())),
        preferred_element_type=jnp.float32,
        precision=lax.Precision.HIGHEST)


def _k5(z):
    return pl.pallas_call(
        _k5_body,
        grid=(pl.cdiv(N, BM), pl.cdiv(N, BN)),
        in_specs=[
            pl.BlockSpec((BM, DZ), lambda i, j: (i, 0)),
            pl.BlockSpec((BN, DZ), lambda i, j: (j, 0)),
        ],
        out_specs=pl.BlockSpec((BM, BN), lambda i, j: (i, j)),
        out_shape=jax.ShapeDtypeStruct((N, N), jnp.float32),
    )(z, z)


def kernel(X, graph, W1, b1, Wmu, bmu, Wsig, bsig, eps):
    pad = jnp.full((EPAD - E,), N, dtype=jnp.int32)
    src = jnp.concatenate([graph[0], pad]).reshape(NC, NS, CH, CB)
    dst = jnp.concatenate([graph[1], pad]).reshape(NC, NS, CH, CB)
    zrow16 = jnp.zeros((RPT, DH), jnp.float32)
    zrow32 = jnp.zeros((RPT, D1), jnp.float32)
    onerow = jnp.concatenate(
        [jnp.ones((CB, 1), jnp.float32), jnp.zeros((CB, DH - 1), jnp.float32)],
        axis=1)

    hist = _sc_hist(dst, onerow, zrow16)
    y1, dinv32 = _k2(X, W1, hist)
    s1 = _sc_agg(src, dst, y1, zrow32)
    y2 = _k3(s1, y1, dinv32, b1.reshape(1, D1))
    s2 = _sc_agg(src, dst, y2, zrow32)
    mus, lsig, z = _k4(s2, y2, dinv32, Wmu, bmu.reshape(1, DZ),
                       Wsig, bsig.reshape(1, DZ), eps)
    zzt = _k5(z)
    return (zzt, mus, lsig)


# speedup vs baseline: 32.7972x; 1.5210x over previous
import functools

import jax
import jax.numpy as jnp
from jax import lax
from jax.experimental import pallas as pl
from jax.experimental.pallas import tpu as pltpu
from jax.experimental.pallas import tpu_sc as plsc

N = 10000
E = 320000
NC, NS = 2, 16
ROWS = 10240
RPT = ROWS // NS
CH, CB = 80, 128
EPW = CH * CB
EPAD = NC * NS * EPW
D1 = 32
DH = 16
DZ = 64

_MESH = plsc.VectorSubcoreMesh(core_axis_name="c", subcore_axis_name="s")
_SC_PARAMS = pltpu.CompilerParams(use_tc_tiling_on_sc=False)


def _hist_body(dst_hbm, onerow_hbm, zrow_hbm, out_hbm, dstv, onesv, acc):
    c = lax.axis_index("c")
    s = lax.axis_index("s")
    pltpu.sync_copy(dst_hbm.at[c, s], dstv)
    pltpu.sync_copy(onerow_hbm, onesv)
    pltpu.sync_copy(zrow_hbm, acc.at[pl.ds(s * RPT, RPT)])
    plsc.subcore_barrier()

    @pl.loop(0, CH)
    def _(j):
        pltpu.sync_copy(onesv, acc.at[dstv.at[j]], add=True)

    plsc.subcore_barrier()
    pltpu.sync_copy(acc.at[pl.ds(s * RPT, RPT)],
                    out_hbm.at[c, pl.ds(s * RPT, RPT)])


@functools.partial(
    pl.kernel,
    out_type=jax.ShapeDtypeStruct((NC, ROWS, DH), jnp.float32),
    mesh=_MESH,
    compiler_params=_SC_PARAMS,
    scratch_types=[
        pltpu.VMEM((CH, CB), jnp.int32),
        pltpu.VMEM((CB, DH), jnp.float32),
        pltpu.VMEM_SHARED((ROWS, DH), jnp.float32),
    ],
)
def _sc_hist(dst_hbm, onerow_hbm, zrow_hbm, out_hbm, dstv, onesv, acc):
    _hist_body(dst_hbm, onerow_hbm, zrow_hbm, out_hbm, dstv, onesv, acc)


def _agg_body(src_hbm, dst_hbm, y_hbm, zrow_hbm, out_hbm, srcv, dstv, rows0,
              rows1, ysh, acc, sem0, sem1):
    c = lax.axis_index("c")
    s = lax.axis_index("s")
    pltpu.sync_copy(src_hbm.at[c, s], srcv)
    pltpu.sync_copy(dst_hbm.at[c, s], dstv)
    pltpu.sync_copy(zrow_hbm, acc.at[pl.ds(s * RPT, RPT)])
    pltpu.sync_copy(y_hbm.at[pl.ds(s * RPT, RPT)], ysh.at[pl.ds(s * RPT, RPT)])
    plsc.subcore_barrier()

    pltpu.async_copy(ysh.at[srcv.at[0]], rows0, sem0)

    @pl.loop(0, CH, step=2)
    def _(j):
        pltpu.async_copy(ysh.at[srcv.at[j + 1]], rows1, sem1)
        pltpu.make_async_copy(y_hbm.at[pl.ds(0, CB)], rows0, sem0).wait()
        pltpu.sync_copy(rows0, acc.at[dstv.at[j]], add=True)

        @pl.when(j + 2 < CH)
        def _():
            pltpu.async_copy(ysh.at[srcv.at[j + 2]], rows0, sem0)

        pltpu.make_async_copy(y_hbm.at[pl.ds(0, CB)], rows1, sem1).wait()
        pltpu.sync_copy(rows1, acc.at[dstv.at[j + 1]], add=True)

    plsc.subcore_barrier()
    pltpu.sync_copy(acc.at[pl.ds(s * RPT, RPT)],
                    out_hbm.at[c, pl.ds(s * RPT, RPT)])


@functools.partial(
    pl.kernel,
    out_type=jax.ShapeDtypeStruct((NC, ROWS, D1), jnp.float32),
    mesh=_MESH,
    compiler_params=_SC_PARAMS,
    scratch_types=[
        pltpu.VMEM((CH, CB), jnp.int32),
        pltpu.VMEM((CH, CB), jnp.int32),
        pltpu.VMEM((CB, D1), jnp.float32),
        pltpu.VMEM((CB, D1), jnp.float32),
        pltpu.VMEM_SHARED((ROWS, D1), jnp.float32),
        pltpu.VMEM_SHARED((ROWS, D1), jnp.float32),
        pltpu.SemaphoreType.DMA,
        pltpu.SemaphoreType.DMA,
    ],
)
def _sc_agg(src_hbm, dst_hbm, y_hbm, zrow_hbm, out_hbm, srcv, dstv, rows0,
            rows1, ysh, acc, sem0, sem1):
    _agg_body(src_hbm, dst_hbm, y_hbm, zrow_hbm, out_hbm, srcv, dstv, rows0,
              rows1, ysh, acc, sem0, sem1)


BLK = 1280


def _k2_body(x_ref, w1_ref, hist_ref, y1_ref, dinv_ref):
    i = pl.program_id(0)
    u = jnp.dot(x_ref[...], w1_ref[...],
                preferred_element_type=jnp.float32,
                precision=lax.Precision.HIGHEST)
    deg = hist_ref[0, :, 0:1] + hist_ref[1, :, 0:1] + 1.0
    rows = i * BLK + lax.broadcasted_iota(jnp.int32, (BLK, 1), 0)
    dinv = jnp.where(rows < N, 1.0 / jnp.sqrt(deg), 0.0)
    dinv32 = jnp.broadcast_to(dinv, (BLK, D1))
    y1_ref[...] = dinv32 * u
    dinv_ref[...] = dinv32


def _k2(X, W1, hist):
    return pl.pallas_call(
        _k2_body,
        grid=(ROWS // BLK,),
        in_specs=[
            pl.BlockSpec((BLK, 128), lambda i: (i, 0)),
            pl.BlockSpec((128, D1), lambda i: (0, 0)),
            pl.BlockSpec((NC, BLK, DH), lambda i: (0, i, 0)),
        ],
        out_specs=[
            pl.BlockSpec((BLK, D1), lambda i: (i, 0)),
            pl.BlockSpec((BLK, D1), lambda i: (i, 0)),
        ],
        out_shape=[
            jax.ShapeDtypeStruct((ROWS, D1), jnp.float32),
            jax.ShapeDtypeStruct((ROWS, D1), jnp.float32),
        ],
    )(X, W1, hist)


def _k3_body(s1_ref, y1_ref, dinv_ref, b1_ref, y2_ref):
    dinv = dinv_ref[...]
    agg = dinv * (s1_ref[0] + s1_ref[1] + y1_ref[...]) + b1_ref[...]
    y2_ref[...] = dinv * jnp.maximum(agg, 0.0)


def _k3(s1, y1, dinv32, b1):
    return pl.pallas_call(
        _k3_body,
        grid=(ROWS // BLK,),
        in_specs=[
            pl.BlockSpec((NC, BLK, D1), lambda i: (0, i, 0)),
            pl.BlockSpec((BLK, D1), lambda i: (i, 0)),
            pl.BlockSpec((BLK, D1), lambda i: (i, 0)),
            pl.BlockSpec((1, D1), lambda i: (0, 0)),
        ],
        out_specs=pl.BlockSpec((BLK, D1), lambda i: (i, 0)),
        out_shape=jax.ShapeDtypeStruct((ROWS, D1), jnp.float32),
    )(s1, y1, dinv32, b1)


def _k4_body(s2_ref, y2_ref, dinv_ref, wmu_ref, bmu_ref, wsig_ref, bsig_ref,
             eps_ref, mus_ref, lsig_ref, z_ref):
    i = pl.program_id(0)
    agg = dinv_ref[...] * (s2_ref[0] + s2_ref[1] + y2_ref[...])
    mus = jnp.dot(agg, wmu_ref[...], preferred_element_type=jnp.float32,
                  precision=lax.Precision.HIGHEST) + bmu_ref[...]
    lsig = jnp.dot(agg, wsig_ref[...], preferred_element_type=jnp.float32,
                   precision=lax.Precision.HIGHEST) + bsig_ref[...]
    z = mus + jnp.exp(0.5 * lsig) * eps_ref[...]
    rows = i * BLK + lax.broadcasted_iota(jnp.int32, (BLK, 1), 0)
    mus_ref[...] = mus
    lsig_ref[...] = lsig
    z_ref[...] = jnp.where(rows < N, z, 0.0)


def _k4(s2, y2, dinv32, Wmu, bmu, Wsig, bsig, eps):
    return pl.pallas_call(
        _k4_body,
        grid=(ROWS // BLK,),
        in_specs=[
            pl.BlockSpec((NC, BLK, D1), lambda i: (0, i, 0)),
            pl.BlockSpec((BLK, D1), lambda i: (i, 0)),
            pl.BlockSpec((BLK, D1), lambda i: (i, 0)),
            pl.BlockSpec((D1, DZ), lambda i: (0, 0)),
            pl.BlockSpec((1, DZ), lambda i: (0, 0)),
            pl.BlockSpec((D1, DZ), lambda i: (0, 0)),
            pl.BlockSpec((1, DZ), lambda i: (0, 0)),
            pl.BlockSpec((BLK, DZ), lambda i: (i, 0)),
        ],
        out_specs=[
            pl.BlockSpec((BLK, DZ), lambda i: (i, 0)),
            pl.BlockSpec((BLK, DZ), lambda i: (i, 0)),
            pl.BlockSpec((BLK, DZ), lambda i: (i, 0)),
        ],
        out_shape=[
            jax.ShapeDtypeStruct((N, DZ), jnp.float32),
            jax.ShapeDtypeStruct((N, DZ), jnp.float32),
            jax.ShapeDtypeStruct((ROWS, DZ), jnp.float32),
        ],
    )(s2, y2, dinv32, Wmu, bmu, Wsig, bsig, eps)


BM, BN = 1024, 2048


def _k5_body(zl_ref, zr_ref, out_ref):
    out_ref[...] = lax.dot_general(
        zl_ref[...], zr_ref[...], (((1,), (1,)), ((), ())),
        preferred_element_type=jnp.float32,
        precision=lax.Precision.DEFAULT)


def _k5(z):
    return pl.pallas_call(
        _k5_body,
        grid=(pl.cdiv(N, BM), pl.cdiv(N, BN)),
        in_specs=[
            pl.BlockSpec((BM, DZ), lambda i, j: (i, 0)),
            pl.BlockSpec((BN, DZ), lambda i, j: (j, 0)),
        ],
        out_specs=pl.BlockSpec((BM, BN), lambda i, j: (i, j)),
        out_shape=jax.ShapeDtypeStruct((N, N), jnp.float32),
    )(z, z)


def kernel(X, graph, W1, b1, Wmu, bmu, Wsig, bsig, eps):
    pad = jnp.full((EPAD - E,), N, dtype=jnp.int32)
    src = jnp.concatenate([graph[0], pad]).reshape(NC, NS, CH, CB)
    dst = jnp.concatenate([graph[1], pad]).reshape(NC, NS, CH, CB)
    zrow16 = jnp.zeros((RPT, DH), jnp.float32)
    zrow32 = jnp.zeros((RPT, D1), jnp.float32)
    onerow = jnp.concatenate(
        [jnp.ones((CB, 1), jnp.float32), jnp.zeros((CB, DH - 1), jnp.float32)],
        axis=1)

    hist = _sc_hist(dst, onerow, zrow16)
    y1, dinv32 = _k2(X, W1, hist)
    s1 = _sc_agg(src, dst, y1, zrow32)
    y2 = _k3(s1, y1, dinv32, b1.reshape(1, D1))
    s2 = _sc_agg(src, dst, y2, zrow32)
    mus, lsig, z = _k4(s2, y2, dinv32, Wmu, bmu.reshape(1, DZ),
                       Wsig, bsig.reshape(1, DZ), eps)
    zzt = _k5(z)
    return (zzt, mus, lsig)


# speedup vs baseline: 33.9986x; 1.0366x over previous
import functools

import jax
import jax.numpy as jnp
from jax import lax
from jax.experimental import pallas as pl
from jax.experimental.pallas import tpu as pltpu
from jax.experimental.pallas import tpu_sc as plsc

N = 10000
E = 320000
NC, NS = 2, 16
ROWS = 10240
RPT = ROWS // NS
CH, CB = 80, 128
EPW = CH * CB
EPAD = NC * NS * EPW
D1 = 32
DH = 16
DZ = 64

_MESH = plsc.VectorSubcoreMesh(core_axis_name="c", subcore_axis_name="s")
_SC_PARAMS = pltpu.CompilerParams(use_tc_tiling_on_sc=False)


def _hist_body(dst_hbm, onerow_hbm, zrow_hbm, out_hbm, dstv, onesv, acc):
    c = lax.axis_index("c")
    s = lax.axis_index("s")
    pltpu.sync_copy(dst_hbm.at[c, s], dstv)
    pltpu.sync_copy(onerow_hbm, onesv)
    pltpu.sync_copy(zrow_hbm, acc.at[pl.ds(s * RPT, RPT)])
    plsc.subcore_barrier()

    @pl.loop(0, CH)
    def _(j):
        pltpu.sync_copy(onesv, acc.at[dstv.at[j]], add=True)

    plsc.subcore_barrier()
    pltpu.sync_copy(acc.at[pl.ds(s * RPT, RPT)],
                    out_hbm.at[c, pl.ds(s * RPT, RPT)])


@functools.partial(
    pl.kernel,
    out_type=jax.ShapeDtypeStruct((NC, ROWS, DH), jnp.float32),
    mesh=_MESH,
    compiler_params=_SC_PARAMS,
    scratch_types=[
        pltpu.VMEM((CH, CB), jnp.int32),
        pltpu.VMEM((CB, DH), jnp.float32),
        pltpu.VMEM_SHARED((ROWS, DH), jnp.float32),
    ],
)
def _sc_hist(dst_hbm, onerow_hbm, zrow_hbm, out_hbm, dstv, onesv, acc):
    _hist_body(dst_hbm, onerow_hbm, zrow_hbm, out_hbm, dstv, onesv, acc)


def _agg_body(src_hbm, dst_hbm, y_hbm, zrow_hbm, out_hbm, srcv, dstv, rows0,
              rows1, ysh, acc, sem0, sem1):
    c = lax.axis_index("c")
    s = lax.axis_index("s")
    pltpu.sync_copy(src_hbm.at[c, s], srcv)
    pltpu.sync_copy(dst_hbm.at[c, s], dstv)
    pltpu.sync_copy(zrow_hbm, acc.at[pl.ds(s * RPT, RPT)])
    pltpu.sync_copy(y_hbm.at[pl.ds(s * RPT, RPT)], ysh.at[pl.ds(s * RPT, RPT)])
    plsc.subcore_barrier()

    pltpu.async_copy(ysh.at[srcv.at[0]], rows0, sem0)

    @pl.loop(0, CH, step=2)
    def _(j):
        pltpu.async_copy(ysh.at[srcv.at[j + 1]], rows1, sem1)
        pltpu.make_async_copy(y_hbm.at[pl.ds(0, CB)], rows0, sem0).wait()
        pltpu.sync_copy(rows0, acc.at[dstv.at[j]], add=True)

        @pl.when(j + 2 < CH)
        def _():
            pltpu.async_copy(ysh.at[srcv.at[j + 2]], rows0, sem0)

        pltpu.make_async_copy(y_hbm.at[pl.ds(0, CB)], rows1, sem1).wait()
        pltpu.sync_copy(rows1, acc.at[dstv.at[j + 1]], add=True)

    plsc.subcore_barrier()
    pltpu.sync_copy(acc.at[pl.ds(s * RPT, RPT)],
                    out_hbm.at[c, pl.ds(s * RPT, RPT)])


@functools.partial(
    pl.kernel,
    out_type=jax.ShapeDtypeStruct((NC, ROWS, D1), jnp.float32),
    mesh=_MESH,
    compiler_params=_SC_PARAMS,
    scratch_types=[
        pltpu.VMEM((CH, CB), jnp.int32),
        pltpu.VMEM((CH, CB), jnp.int32),
        pltpu.VMEM((CB, D1), jnp.float32),
        pltpu.VMEM((CB, D1), jnp.float32),
        pltpu.VMEM_SHARED((ROWS, D1), jnp.float32),
        pltpu.VMEM_SHARED((ROWS, D1), jnp.float32),
        pltpu.SemaphoreType.DMA,
        pltpu.SemaphoreType.DMA,
    ],
)
def _sc_agg(src_hbm, dst_hbm, y_hbm, zrow_hbm, out_hbm, srcv, dstv, rows0,
            rows1, ysh, acc, sem0, sem1):
    _agg_body(src_hbm, dst_hbm, y_hbm, zrow_hbm, out_hbm, srcv, dstv, rows0,
              rows1, ysh, acc, sem0, sem1)


BLK = 1280


def _k2_body(x_ref, w1_ref, hist_ref, y1_ref, dinv_ref):
    i = pl.program_id(0)
    u = jnp.dot(x_ref[...], w1_ref[...],
                preferred_element_type=jnp.float32,
                precision=lax.Precision.HIGHEST)
    deg = hist_ref[0, :, 0:1] + hist_ref[1, :, 0:1] + 1.0
    rows = i * BLK + lax.broadcasted_iota(jnp.int32, (BLK, 1), 0)
    dinv = jnp.where(rows < N, 1.0 / jnp.sqrt(deg), 0.0)
    dinv32 = jnp.broadcast_to(dinv, (BLK, D1))
    y1_ref[...] = dinv32 * u
    dinv_ref[...] = dinv32


def _k2(X, W1, hist):
    return pl.pallas_call(
        _k2_body,
        grid=(ROWS // BLK,),
        in_specs=[
            pl.BlockSpec((BLK, 128), lambda i: (i, 0)),
            pl.BlockSpec((128, D1), lambda i: (0, 0)),
            pl.BlockSpec((NC, BLK, DH), lambda i: (0, i, 0)),
        ],
        out_specs=[
            pl.BlockSpec((BLK, D1), lambda i: (i, 0)),
            pl.BlockSpec((BLK, D1), lambda i: (i, 0)),
        ],
        out_shape=[
            jax.ShapeDtypeStruct((ROWS, D1), jnp.float32),
            jax.ShapeDtypeStruct((ROWS, D1), jnp.float32),
        ],
    )(X, W1, hist)


def _k3_body(s1_ref, y1_ref, dinv_ref, b1_ref, y2_ref):
    dinv = dinv_ref[...]
    agg = dinv * (s1_ref[0] + s1_ref[1] + y1_ref[...]) + b1_ref[...]
    y2_ref[...] = dinv * jnp.maximum(agg, 0.0)


def _k3(s1, y1, dinv32, b1):
    return pl.pallas_call(
        _k3_body,
        grid=(ROWS // BLK,),
        in_specs=[
            pl.BlockSpec((NC, BLK, D1), lambda i: (0, i, 0)),
            pl.BlockSpec((BLK, D1), lambda i: (i, 0)),
            pl.BlockSpec((BLK, D1), lambda i: (i, 0)),
            pl.BlockSpec((1, D1), lambda i: (0, 0)),
        ],
        out_specs=pl.BlockSpec((BLK, D1), lambda i: (i, 0)),
        out_shape=jax.ShapeDtypeStruct((ROWS, D1), jnp.float32),
    )(s1, y1, dinv32, b1)


def _k4_body(s2_ref, y2_ref, dinv_ref, wmu_ref, bmu_ref, wsig_ref, bsig_ref,
             eps_ref, mus_ref, lsig_ref, z_ref):
    i = pl.program_id(0)
    agg = dinv_ref[...] * (s2_ref[0] + s2_ref[1] + y2_ref[...])
    mus = jnp.dot(agg, wmu_ref[...], preferred_element_type=jnp.float32,
                  precision=lax.Precision.HIGHEST) + bmu_ref[...]
    lsig = jnp.dot(agg, wsig_ref[...], preferred_element_type=jnp.float32,
                   precision=lax.Precision.HIGHEST) + bsig_ref[...]
    z = mus + jnp.exp(0.5 * lsig) * eps_ref[...]
    rows = i * BLK + lax.broadcasted_iota(jnp.int32, (BLK, 1), 0)
    mus_ref[...] = mus
    lsig_ref[...] = lsig
    z_ref[...] = jnp.where(rows < N, z, 0.0)


def _k4(s2, y2, dinv32, Wmu, bmu, Wsig, bsig, eps):
    return pl.pallas_call(
        _k4_body,
        grid=(ROWS // BLK,),
        in_specs=[
            pl.BlockSpec((NC, BLK, D1), lambda i: (0, i, 0)),
            pl.BlockSpec((BLK, D1), lambda i: (i, 0)),
            pl.BlockSpec((BLK, D1), lambda i: (i, 0)),
            pl.BlockSpec((D1, DZ), lambda i: (0, 0)),
            pl.BlockSpec((1, DZ), lambda i: (0, 0)),
            pl.BlockSpec((D1, DZ), lambda i: (0, 0)),
            pl.BlockSpec((1, DZ), lambda i: (0, 0)),
            pl.BlockSpec((BLK, DZ), lambda i: (i, 0)),
        ],
        out_specs=[
            pl.BlockSpec((BLK, DZ), lambda i: (i, 0)),
            pl.BlockSpec((BLK, DZ), lambda i: (i, 0)),
            pl.BlockSpec((BLK, DZ), lambda i: (i, 0)),
        ],
        out_shape=[
            jax.ShapeDtypeStruct((N, DZ), jnp.float32),
            jax.ShapeDtypeStruct((N, DZ), jnp.float32),
            jax.ShapeDtypeStruct((ROWS, DZ), jnp.float32),
        ],
    )(s2, y2, dinv32, Wmu, bmu, Wsig, bsig, eps)


BM, BN = 2048, 2048


def _k5_body(zl_ref, zr_ref, out_ref):
    out_ref[...] = lax.dot_general(
        zl_ref[...], zr_ref[...], (((1,), (1,)), ((), ())),
        preferred_element_type=jnp.float32,
        precision=lax.Precision.DEFAULT)


def _k5(z):
    return pl.pallas_call(
        _k5_body,
        grid=(pl.cdiv(N, BM), pl.cdiv(N, BN)),
        in_specs=[
            pl.BlockSpec((BM, DZ), lambda i, j: (i, 0)),
            pl.BlockSpec((BN, DZ), lambda i, j: (j, 0)),
        ],
        out_specs=pl.BlockSpec((BM, BN), lambda i, j: (i, j)),
        out_shape=jax.ShapeDtypeStruct((N, N), jnp.float32),
    )(z, z)


def kernel(X, graph, W1, b1, Wmu, bmu, Wsig, bsig, eps):
    pad = jnp.full((EPAD - E,), N, dtype=jnp.int32)
    src = jnp.concatenate([graph[0], pad]).reshape(NC, NS, CH, CB)
    dst = jnp.concatenate([graph[1], pad]).reshape(NC, NS, CH, CB)
    zrow16 = jnp.zeros((RPT, DH), jnp.float32)
    zrow32 = jnp.zeros((RPT, D1), jnp.float32)
    onerow = jnp.concatenate(
        [jnp.ones((CB, 1), jnp.float32), jnp.zeros((CB, DH - 1), jnp.float32)],
        axis=1)

    hist = _sc_hist(dst, onerow, zrow16)
    y1, dinv32 = _k2(X, W1, hist)
    s1 = _sc_agg(src, dst, y1, zrow32)
    y2 = _k3(s1, y1, dinv32, b1.reshape(1, D1))
    s2 = _sc_agg(src, dst, y2, zrow32)
    mus, lsig, z = _k4(s2, y2, dinv32, Wmu, bmu.reshape(1, DZ),
                       Wsig, bsig.reshape(1, DZ), eps)
    zzt = _k5(z)
    return (zzt, mus, lsig)


# speedup vs baseline: 34.1388x; 1.0041x over previous
import functools

import jax
import jax.numpy as jnp
from jax import lax
from jax.experimental import pallas as pl
from jax.experimental.pallas import tpu as pltpu
from jax.experimental.pallas import tpu_sc as plsc

N = 10000
E = 320000
NC, NS = 2, 16
ROWS = 10240
RPT = ROWS // NS
CH, CB = 80, 128
EPW = CH * CB
EPAD = NC * NS * EPW
D1 = 32
DH = 16
DZ = 64

_MESH = plsc.VectorSubcoreMesh(core_axis_name="c", subcore_axis_name="s")
_SC_PARAMS = pltpu.CompilerParams(use_tc_tiling_on_sc=False)


def _hist_body(dst_hbm, onerow_hbm, zrow_hbm, out_hbm, dstv, onesv, acc):
    c = lax.axis_index("c")
    s = lax.axis_index("s")
    pltpu.sync_copy(dst_hbm.at[c, s], dstv)
    pltpu.sync_copy(onerow_hbm, onesv)
    pltpu.sync_copy(zrow_hbm, acc.at[pl.ds(s * RPT, RPT)])
    plsc.subcore_barrier()

    @pl.loop(0, CH)
    def _(j):
        pltpu.sync_copy(onesv, acc.at[dstv.at[j]], add=True)

    plsc.subcore_barrier()
    pltpu.sync_copy(acc.at[pl.ds(s * RPT, RPT)],
                    out_hbm.at[c, pl.ds(s * RPT, RPT)])


@functools.partial(
    pl.kernel,
    out_type=jax.ShapeDtypeStruct((NC, ROWS, DH), jnp.float32),
    mesh=_MESH,
    compiler_params=_SC_PARAMS,
    scratch_types=[
        pltpu.VMEM((CH, CB), jnp.int32),
        pltpu.VMEM((CB, DH), jnp.float32),
        pltpu.VMEM_SHARED((ROWS, DH), jnp.float32),
    ],
)
def _sc_hist(dst_hbm, onerow_hbm, zrow_hbm, out_hbm, dstv, onesv, acc):
    _hist_body(dst_hbm, onerow_hbm, zrow_hbm, out_hbm, dstv, onesv, acc)


def _agg_body(src_hbm, dst_hbm, y_hbm, zrow_hbm, out_hbm, srcv, dstv, rows0,
              rows1, ysh, acc, sem0, sem1):
    c = lax.axis_index("c")
    s = lax.axis_index("s")
    pltpu.sync_copy(src_hbm.at[c, s], srcv)
    pltpu.sync_copy(dst_hbm.at[c, s], dstv)
    pltpu.sync_copy(zrow_hbm, acc.at[pl.ds(s * RPT, RPT)])
    pltpu.sync_copy(y_hbm.at[pl.ds(s * RPT, RPT)], ysh.at[pl.ds(s * RPT, RPT)])
    plsc.subcore_barrier()

    pltpu.async_copy(ysh.at[srcv.at[0]], rows0, sem0)

    @pl.loop(0, CH, step=2)
    def _(j):
        pltpu.async_copy(ysh.at[srcv.at[j + 1]], rows1, sem1)
        pltpu.make_async_copy(y_hbm.at[pl.ds(0, CB)], rows0, sem0).wait()
        pltpu.sync_copy(rows0, acc.at[dstv.at[j]], add=True)

        @pl.when(j + 2 < CH)
        def _():
            pltpu.async_copy(ysh.at[srcv.at[j + 2]], rows0, sem0)

        pltpu.make_async_copy(y_hbm.at[pl.ds(0, CB)], rows1, sem1).wait()
        pltpu.sync_copy(rows1, acc.at[dstv.at[j + 1]], add=True)

    plsc.subcore_barrier()
    pltpu.sync_copy(acc.at[pl.ds(s * RPT, RPT)],
                    out_hbm.at[c, pl.ds(s * RPT, RPT)])


@functools.partial(
    pl.kernel,
    out_type=jax.ShapeDtypeStruct((NC, ROWS, D1), jnp.float32),
    mesh=_MESH,
    compiler_params=_SC_PARAMS,
    scratch_types=[
        pltpu.VMEM((CH, CB), jnp.int32),
        pltpu.VMEM((CH, CB), jnp.int32),
        pltpu.VMEM((CB, D1), jnp.float32),
        pltpu.VMEM((CB, D1), jnp.float32),
        pltpu.VMEM_SHARED((ROWS, D1), jnp.float32),
        pltpu.VMEM_SHARED((ROWS, D1), jnp.float32),
        pltpu.SemaphoreType.DMA,
        pltpu.SemaphoreType.DMA,
    ],
)
def _sc_agg(src_hbm, dst_hbm, y_hbm, zrow_hbm, out_hbm, srcv, dstv, rows0,
            rows1, ysh, acc, sem0, sem1):
    _agg_body(src_hbm, dst_hbm, y_hbm, zrow_hbm, out_hbm, srcv, dstv, rows0,
              rows1, ysh, acc, sem0, sem1)


BLK = 1280


def _k2_body(x_ref, w1_ref, hist_ref, y1_ref, dinv_ref):
    i = pl.program_id(0)
    u = jnp.dot(x_ref[...], w1_ref[...],
                preferred_element_type=jnp.float32,
                precision=lax.Precision.HIGHEST)
    deg = hist_ref[0, :, 0:1] + hist_ref[1, :, 0:1] + 1.0
    rows = i * BLK + lax.broadcasted_iota(jnp.int32, (BLK, 1), 0)
    dinv = jnp.where(rows < N, 1.0 / jnp.sqrt(deg), 0.0)
    dinv32 = jnp.broadcast_to(dinv, (BLK, D1))
    y1_ref[...] = dinv32 * u
    dinv_ref[...] = dinv32


def _k2(X, W1, hist):
    return pl.pallas_call(
        _k2_body,
        grid=(ROWS // BLK,),
        in_specs=[
            pl.BlockSpec((BLK, 128), lambda i: (i, 0)),
            pl.BlockSpec((128, D1), lambda i: (0, 0)),
            pl.BlockSpec((NC, BLK, DH), lambda i: (0, i, 0)),
        ],
        out_specs=[
            pl.BlockSpec((BLK, D1), lambda i: (i, 0)),
            pl.BlockSpec((BLK, D1), lambda i: (i, 0)),
        ],
        out_shape=[
            jax.ShapeDtypeStruct((ROWS, D1), jnp.float32),
            jax.ShapeDtypeStruct((ROWS, D1), jnp.float32),
        ],
    )(X, W1, hist)


def _k3_body(s1_ref, y1_ref, dinv_ref, b1_ref, y2_ref):
    dinv = dinv_ref[...]
    agg = dinv * (s1_ref[0] + s1_ref[1] + y1_ref[...]) + b1_ref[...]
    y2_ref[...] = dinv * jnp.maximum(agg, 0.0)


def _k3(s1, y1, dinv32, b1):
    return pl.pallas_call(
        _k3_body,
        grid=(ROWS // BLK,),
        in_specs=[
            pl.BlockSpec((NC, BLK, D1), lambda i: (0, i, 0)),
            pl.BlockSpec((BLK, D1), lambda i: (i, 0)),
            pl.BlockSpec((BLK, D1), lambda i: (i, 0)),
            pl.BlockSpec((1, D1), lambda i: (0, 0)),
        ],
        out_specs=pl.BlockSpec((BLK, D1), lambda i: (i, 0)),
        out_shape=jax.ShapeDtypeStruct((ROWS, D1), jnp.float32),
    )(s1, y1, dinv32, b1)


def _k4_body(s2_ref, y2_ref, dinv_ref, wmu_ref, bmu_ref, wsig_ref, bsig_ref,
             eps_ref, mus_ref, lsig_ref, z_ref):
    i = pl.program_id(0)
    agg = dinv_ref[...] * (s2_ref[0] + s2_ref[1] + y2_ref[...])
    mus = jnp.dot(agg, wmu_ref[...], preferred_element_type=jnp.float32,
                  precision=lax.Precision.HIGHEST) + bmu_ref[...]
    lsig = jnp.dot(agg, wsig_ref[...], preferred_element_type=jnp.float32,
                   precision=lax.Precision.HIGHEST) + bsig_ref[...]
    z = mus + jnp.exp(0.5 * lsig) * eps_ref[...]
    rows = i * BLK + lax.broadcasted_iota(jnp.int32, (BLK, 1), 0)
    mus_ref[...] = mus
    lsig_ref[...] = lsig
    z_ref[...] = jnp.where(rows < N, z, 0.0)


def _k4(s2, y2, dinv32, Wmu, bmu, Wsig, bsig, eps):
    return pl.pallas_call(
        _k4_body,
        grid=(ROWS // BLK,),
        in_specs=[
            pl.BlockSpec((NC, BLK, D1), lambda i: (0, i, 0)),
            pl.BlockSpec((BLK, D1), lambda i: (i, 0)),
            pl.BlockSpec((BLK, D1), lambda i: (i, 0)),
            pl.BlockSpec((D1, DZ), lambda i: (0, 0)),
            pl.BlockSpec((1, DZ), lambda i: (0, 0)),
            pl.BlockSpec((D1, DZ), lambda i: (0, 0)),
            pl.BlockSpec((1, DZ), lambda i: (0, 0)),
            pl.BlockSpec((BLK, DZ), lambda i: (i, 0)),
        ],
        out_specs=[
            pl.BlockSpec((BLK, DZ), lambda i: (i, 0)),
            pl.BlockSpec((BLK, DZ), lambda i: (i, 0)),
            pl.BlockSpec((BLK, DZ), lambda i: (i, 0)),
        ],
        out_shape=[
            jax.ShapeDtypeStruct((N, DZ), jnp.float32),
            jax.ShapeDtypeStruct((N, DZ), jnp.float32),
            jax.ShapeDtypeStruct((ROWS, DZ), jnp.float32),
        ],
    )(s2, y2, dinv32, Wmu, bmu, Wsig, bsig, eps)


BM, BN = 2560, 2560


def _k5_body(zl_ref, zr_ref, out_ref):
    out_ref[...] = lax.dot_general(
        zl_ref[...], zr_ref[...], (((1,), (1,)), ((), ())),
        preferred_element_type=jnp.float32,
        precision=lax.Precision.DEFAULT)


def _k5(z):
    return pl.pallas_call(
        _k5_body,
        grid=(pl.cdiv(N, BM), pl.cdiv(N, BN)),
        in_specs=[
            pl.BlockSpec((BM, DZ), lambda i, j: (i, 0)),
            pl.BlockSpec((BN, DZ), lambda i, j: (j, 0)),
        ],
        out_specs=pl.BlockSpec((BM, BN), lambda i, j: (i, j)),
        out_shape=jax.ShapeDtypeStruct((N, N), jnp.float32),
    )(z, z)


def kernel(X, graph, W1, b1, Wmu, bmu, Wsig, bsig, eps):
    pad = jnp.full((EPAD - E,), N, dtype=jnp.int32)
    src = jnp.concatenate([graph[0], pad]).reshape(NC, NS, CH, CB)
    dst = jnp.concatenate([graph[1], pad]).reshape(NC, NS, CH, CB)
    zrow16 = jnp.zeros((RPT, DH), jnp.float32)
    zrow32 = jnp.zeros((RPT, D1), jnp.float32)
    onerow = jnp.concatenate(
        [jnp.ones((CB, 1), jnp.float32), jnp.zeros((CB, DH - 1), jnp.float32)],
        axis=1)

    hist = _sc_hist(dst, onerow, zrow16)
    y1, dinv32 = _k2(X, W1, hist)
    s1 = _sc_agg(src, dst, y1, zrow32)
    y2 = _k3(s1, y1, dinv32, b1.reshape(1, D1))
    s2 = _sc_agg(src, dst, y2, zrow32)
    mus, lsig, z = _k4(s2, y2, dinv32, Wmu, bmu.reshape(1, DZ),
                       Wsig, bsig.reshape(1, DZ), eps)
    zzt = _k5(z)
    return (zzt, mus, lsig)


# speedup vs baseline: 35.6209x; 1.0434x over previous
import functools

import jax
import jax.numpy as jnp
from jax import lax
from jax.experimental import pallas as pl
from jax.experimental.pallas import tpu as pltpu
from jax.experimental.pallas import tpu_sc as plsc

N = 10000
E = 320000
NC, NS = 2, 16
NW = NC * NS
ROWS = 10240
RPT = ROWS // NS
CH, CB = 125, 80
D1 = 32
DH = 16
DZ = 64

_MESH = plsc.VectorSubcoreMesh(core_axis_name="c", subcore_axis_name="s")
_SC_PARAMS = pltpu.CompilerParams(use_tc_tiling_on_sc=False)


def _hist_body(g_hbm, onerow_hbm, zrow_hbm, out_hbm, dstv, onesv, acc):
    c = lax.axis_index("c")
    s = lax.axis_index("s")
    pltpu.sync_copy(g_hbm.at[1, c * NS + s], dstv)
    pltpu.sync_copy(onerow_hbm, onesv)
    pltpu.sync_copy(zrow_hbm, acc.at[pl.ds(s * RPT, RPT)])
    plsc.subcore_barrier()

    @pl.loop(0, CH)
    def _(j):
        pltpu.sync_copy(onesv, acc.at[dstv.at[j]], add=True)

    plsc.subcore_barrier()
    pltpu.sync_copy(acc.at[pl.ds(s * RPT, RPT)],
                    out_hbm.at[c, pl.ds(s * RPT, RPT)])


@functools.partial(
    pl.kernel,
    out_type=jax.ShapeDtypeStruct((NC, ROWS, DH), jnp.float32),
    mesh=_MESH,
    compiler_params=_SC_PARAMS,
    scratch_types=[
        pltpu.VMEM((CH, CB), jnp.int32),
        pltpu.VMEM((CB, DH), jnp.float32),
        pltpu.VMEM_SHARED((ROWS, DH), jnp.float32),
    ],
)
def _sc_hist(g_hbm, onerow_hbm, zrow_hbm, out_hbm, dstv, onesv, acc):
    _hist_body(g_hbm, onerow_hbm, zrow_hbm, out_hbm, dstv, onesv, acc)


def _agg_body(g_hbm, y_hbm, zrow_hbm, out_hbm, srcv, dstv, rows0,
              rows1, ysh, acc, sem0, sem1):
    c = lax.axis_index("c")
    s = lax.axis_index("s")
    pltpu.sync_copy(g_hbm.at[0, c * NS + s], srcv)
    pltpu.sync_copy(g_hbm.at[1, c * NS + s], dstv)
    pltpu.sync_copy(zrow_hbm, acc.at[pl.ds(s * RPT, RPT)])
    pltpu.sync_copy(y_hbm.at[pl.ds(s * RPT, RPT)], ysh.at[pl.ds(s * RPT, RPT)])
    plsc.subcore_barrier()

    pltpu.async_copy(ysh.at[srcv.at[0]], rows0, sem0)

    @pl.loop(0, CH - 1, step=2)
    def _(j):
        pltpu.async_copy(ysh.at[srcv.at[j + 1]], rows1, sem1)
        pltpu.make_async_copy(y_hbm.at[pl.ds(0, CB)], rows0, sem0).wait()
        pltpu.sync_copy(rows0, acc.at[dstv.at[j]], add=True)
        pltpu.async_copy(ysh.at[srcv.at[j + 2]], rows0, sem0)
        pltpu.make_async_copy(y_hbm.at[pl.ds(0, CB)], rows1, sem1).wait()
        pltpu.sync_copy(rows1, acc.at[dstv.at[j + 1]], add=True)

    pltpu.make_async_copy(y_hbm.at[pl.ds(0, CB)], rows0, sem0).wait()
    pltpu.sync_copy(rows0, acc.at[dstv.at[CH - 1]], add=True)

    plsc.subcore_barrier()
    pltpu.sync_copy(acc.at[pl.ds(s * RPT, RPT)],
                    out_hbm.at[c, pl.ds(s * RPT, RPT)])


@functools.partial(
    pl.kernel,
    out_type=jax.ShapeDtypeStruct((NC, ROWS, D1), jnp.float32),
    mesh=_MESH,
    compiler_params=_SC_PARAMS,
    scratch_types=[
        pltpu.VMEM((CH, CB), jnp.int32),
        pltpu.VMEM((CH, CB), jnp.int32),
        pltpu.VMEM((CB, D1), jnp.float32),
        pltpu.VMEM((CB, D1), jnp.float32),
        pltpu.VMEM_SHARED((ROWS, D1), jnp.float32),
        pltpu.VMEM_SHARED((ROWS, D1), jnp.float32),
        pltpu.SemaphoreType.DMA,
        pltpu.SemaphoreType.DMA,
    ],
)
def _sc_agg(g_hbm, y_hbm, zrow_hbm, out_hbm, srcv, dstv, rows0,
            rows1, ysh, acc, sem0, sem1):
    _agg_body(g_hbm, y_hbm, zrow_hbm, out_hbm, srcv, dstv, rows0,
              rows1, ysh, acc, sem0, sem1)


BLK = 1280


def _k2_body(x_ref, w1_ref, hist_ref, y1_ref, dinv_ref):
    i = pl.program_id(0)
    u = jnp.dot(x_ref[...], w1_ref[...],
                preferred_element_type=jnp.float32,
                precision=lax.Precision.HIGHEST)
    deg = hist_ref[0, :, 0:1] + hist_ref[1, :, 0:1] + 1.0
    rows = i * BLK + lax.broadcasted_iota(jnp.int32, (BLK, 1), 0)
    dinv = jnp.where(rows < N, 1.0 / jnp.sqrt(deg), 0.0)
    dinv32 = jnp.broadcast_to(dinv, (BLK, D1))
    y1_ref[...] = dinv32 * u
    dinv_ref[...] = dinv32


def _k2(X, W1, hist):
    return pl.pallas_call(
        _k2_body,
        grid=(ROWS // BLK,),
        in_specs=[
            pl.BlockSpec((BLK, 128), lambda i: (i, 0)),
            pl.BlockSpec((128, D1), lambda i: (0, 0)),
            pl.BlockSpec((NC, BLK, DH), lambda i: (0, i, 0)),
        ],
        out_specs=[
            pl.BlockSpec((BLK, D1), lambda i: (i, 0)),
            pl.BlockSpec((BLK, D1), lambda i: (i, 0)),
        ],
        out_shape=[
            jax.ShapeDtypeStruct((ROWS, D1), jnp.float32),
            jax.ShapeDtypeStruct((ROWS, D1), jnp.float32),
        ],
    )(X, W1, hist)


def _k3_body(s1_ref, y1_ref, dinv_ref, b1_ref, y2_ref):
    dinv = dinv_ref[...]
    agg = dinv * (s1_ref[0] + s1_ref[1] + y1_ref[...]) + b1_ref[...]
    y2_ref[...] = dinv * jnp.maximum(agg, 0.0)


def _k3(s1, y1, dinv32, b1):
    return pl.pallas_call(
        _k3_body,
        grid=(ROWS // BLK,),
        in_specs=[
            pl.BlockSpec((NC, BLK, D1), lambda i: (0, i, 0)),
            pl.BlockSpec((BLK, D1), lambda i: (i, 0)),
            pl.BlockSpec((BLK, D1), lambda i: (i, 0)),
            pl.BlockSpec((1, D1), lambda i: (0, 0)),
        ],
        out_specs=pl.BlockSpec((BLK, D1), lambda i: (i, 0)),
        out_shape=jax.ShapeDtypeStruct((ROWS, D1), jnp.float32),
    )(s1, y1, dinv32, b1)


def _k4_body(s2_ref, y2_ref, dinv_ref, wmu_ref, bmu_ref, wsig_ref, bsig_ref,
             eps_ref, mus_ref, lsig_ref, z_ref):
    i = pl.program_id(0)
    agg = dinv_ref[...] * (s2_ref[0] + s2_ref[1] + y2_ref[...])
    mus = jnp.dot(agg, wmu_ref[...], preferred_element_type=jnp.float32,
                  precision=lax.Precision.HIGHEST) + bmu_ref[...]
    lsig = jnp.dot(agg, wsig_ref[...], preferred_element_type=jnp.float32,
                   precision=lax.Precision.HIGHEST) + bsig_ref[...]
    z = mus + jnp.exp(0.5 * lsig) * eps_ref[...]
    rows = i * BLK + lax.broadcasted_iota(jnp.int32, (BLK, 1), 0)
    mus_ref[...] = mus
    lsig_ref[...] = lsig
    z_ref[...] = jnp.where(rows < N, z, 0.0)


def _k4(s2, y2, dinv32, Wmu, bmu, Wsig, bsig, eps):
    return pl.pallas_call(
        _k4_body,
        grid=(ROWS // BLK,),
        in_specs=[
            pl.BlockSpec((NC, BLK, D1), lambda i: (0, i, 0)),
            pl.BlockSpec((BLK, D1), lambda i: (i, 0)),
            pl.BlockSpec((BLK, D1), lambda i: (i, 0)),
            pl.BlockSpec((D1, DZ), lambda i: (0, 0)),
            pl.BlockSpec((1, DZ), lambda i: (0, 0)),
            pl.BlockSpec((D1, DZ), lambda i: (0, 0)),
            pl.BlockSpec((1, DZ), lambda i: (0, 0)),
            pl.BlockSpec((BLK, DZ), lambda i: (i, 0)),
        ],
        out_specs=[
            pl.BlockSpec((BLK, DZ), lambda i: (i, 0)),
            pl.BlockSpec((BLK, DZ), lambda i: (i, 0)),
            pl.BlockSpec((BLK, DZ), lambda i: (i, 0)),
        ],
        out_shape=[
            jax.ShapeDtypeStruct((N, DZ), jnp.float32),
            jax.ShapeDtypeStruct((N, DZ), jnp.float32),
            jax.ShapeDtypeStruct((ROWS, DZ), jnp.float32),
        ],
    )(s2, y2, dinv32, Wmu, bmu, Wsig, bsig, eps)


BM, BN = 2560, 2560


def _k5_body(zl_ref, zr_ref, out_ref):
    out_ref[...] = lax.dot_general(
        zl_ref[...], zr_ref[...], (((1,), (1,)), ((), ())),
        preferred_element_type=jnp.float32,
        precision=lax.Precision.DEFAULT)


def _k5(z):
    return pl.pallas_call(
        _k5_body,
        grid=(pl.cdiv(N, BM), pl.cdiv(N, BN)),
        in_specs=[
            pl.BlockSpec((BM, DZ), lambda i, j: (i, 0)),
            pl.BlockSpec((BN, DZ), lambda i, j: (j, 0)),
        ],
        out_specs=pl.BlockSpec((BM, BN), lambda i, j: (i, j)),
        out_shape=jax.ShapeDtypeStruct((N, N), jnp.float32),
    )(z, z)


def kernel(X, graph, W1, b1, Wmu, bmu, Wsig, bsig, eps):
    g = graph.reshape(2, NW, CH, CB)
    zrow16 = jnp.zeros((RPT, DH), jnp.float32)
    zrow32 = jnp.zeros((RPT, D1), jnp.float32)
    onerow = jnp.concatenate(
        [jnp.ones((CB, 1), jnp.float32), jnp.zeros((CB, DH - 1), jnp.float32)],
        axis=1)

    hist = _sc_hist(g, onerow, zrow16)
    y1, dinv32 = _k2(X, W1, hist)
    s1 = _sc_agg(g, y1, zrow32)
    y2 = _k3(s1, y1, dinv32, b1.reshape(1, D1))
    s2 = _sc_agg(g, y2, zrow32)
    mus, lsig, z = _k4(s2, y2, dinv32, Wmu, bmu.reshape(1, DZ),
                       Wsig, bsig.reshape(1, DZ), eps)
    zzt = _k5(z)
    return (zzt, mus, lsig)


# speedup vs baseline: 36.0496x; 1.0120x over previous
import functools

import jax
import jax.numpy as jnp
from jax import lax
from jax.experimental import pallas as pl
from jax.experimental.pallas import tpu as pltpu
from jax.experimental.pallas import tpu_sc as plsc

N = 10000
E = 320000
NC, NS = 2, 16
NW = NC * NS
ROWS = 10240
RPT = ROWS // NS
CH, CB = 125, 80
D1 = 32
DH = 16
DZ = 64

_MESH = plsc.VectorSubcoreMesh(core_axis_name="c", subcore_axis_name="s")
_SC_PARAMS = pltpu.CompilerParams(use_tc_tiling_on_sc=False)


def _hist_body(g_hbm, onerow_hbm, zrow_hbm, out_hbm, dstv, onesv, acc):
    c = lax.axis_index("c")
    s = lax.axis_index("s")
    pltpu.sync_copy(g_hbm.at[1, c * NS + s], dstv)
    pltpu.sync_copy(onerow_hbm, onesv)
    pltpu.sync_copy(zrow_hbm, acc.at[pl.ds(s * RPT, RPT)])
    plsc.subcore_barrier()

    @pl.loop(0, CH)
    def _(j):
        pltpu.sync_copy(onesv, acc.at[dstv.at[j]], add=True)

    plsc.subcore_barrier()
    pltpu.sync_copy(acc.at[pl.ds(s * RPT, RPT)],
                    out_hbm.at[c, pl.ds(s * RPT, RPT)])


@functools.partial(
    pl.kernel,
    out_type=jax.ShapeDtypeStruct((NC, ROWS, DH), jnp.float32),
    mesh=_MESH,
    compiler_params=_SC_PARAMS,
    scratch_types=[
        pltpu.VMEM((CH, CB), jnp.int32),
        pltpu.VMEM((CB, DH), jnp.float32),
        pltpu.VMEM_SHARED((ROWS, DH), jnp.float32),
    ],
)
def _sc_hist(g_hbm, onerow_hbm, zrow_hbm, out_hbm, dstv, onesv, acc):
    _hist_body(g_hbm, onerow_hbm, zrow_hbm, out_hbm, dstv, onesv, acc)


def _agg_body(g_hbm, y_hbm, zrow_hbm, out_hbm, srcv, dstv, rows0,
              rows1, ysh, acc, sem0, sem1):
    c = lax.axis_index("c")
    s = lax.axis_index("s")
    pltpu.sync_copy(g_hbm.at[0, c * NS + s], srcv)
    pltpu.sync_copy(g_hbm.at[1, c * NS + s], dstv)
    pltpu.sync_copy(zrow_hbm, acc.at[pl.ds(s * RPT, RPT)])
    pltpu.sync_copy(y_hbm.at[pl.ds(s * RPT, RPT)], ysh.at[pl.ds(s * RPT, RPT)])
    plsc.subcore_barrier()

    pltpu.async_copy(ysh.at[srcv.at[0]], rows0, sem0)

    @pl.loop(0, CH - 1, step=2)
    def _(j):
        pltpu.async_copy(ysh.at[srcv.at[j + 1]], rows1, sem1)
        pltpu.make_async_copy(y_hbm.at[pl.ds(0, CB)], rows0, sem0).wait()
        pltpu.sync_copy(rows0, acc.at[dstv.at[j]], add=True)
        pltpu.async_copy(ysh.at[srcv.at[j + 2]], rows0, sem0)
        pltpu.make_async_copy(y_hbm.at[pl.ds(0, CB)], rows1, sem1).wait()
        pltpu.sync_copy(rows1, acc.at[dstv.at[j + 1]], add=True)

    pltpu.make_async_copy(y_hbm.at[pl.ds(0, CB)], rows0, sem0).wait()
    pltpu.sync_copy(rows0, acc.at[dstv.at[CH - 1]], add=True)

    plsc.subcore_barrier()
    pltpu.sync_copy(acc.at[pl.ds(s * RPT, RPT)],
                    out_hbm.at[c, pl.ds(s * RPT, RPT)])


@functools.partial(
    pl.kernel,
    out_type=jax.ShapeDtypeStruct((NC, ROWS, D1), jnp.float32),
    mesh=_MESH,
    compiler_params=_SC_PARAMS,
    scratch_types=[
        pltpu.VMEM((CH, CB), jnp.int32),
        pltpu.VMEM((CH, CB), jnp.int32),
        pltpu.VMEM((CB, D1), jnp.float32),
        pltpu.VMEM((CB, D1), jnp.float32),
        pltpu.VMEM_SHARED((ROWS, D1), jnp.float32),
        pltpu.VMEM_SHARED((ROWS, D1), jnp.float32),
        pltpu.SemaphoreType.DMA,
        pltpu.SemaphoreType.DMA,
    ],
)
def _sc_agg(g_hbm, y_hbm, zrow_hbm, out_hbm, srcv, dstv, rows0,
            rows1, ysh, acc, sem0, sem1):
    _agg_body(g_hbm, y_hbm, zrow_hbm, out_hbm, srcv, dstv, rows0,
              rows1, ysh, acc, sem0, sem1)


BLK = 2560


def _k2_body(x_ref, w1_ref, hist_ref, y1_ref, dinv_ref):
    i = pl.program_id(0)
    u = jnp.dot(x_ref[...], w1_ref[...],
                preferred_element_type=jnp.float32,
                precision=lax.Precision.HIGHEST)
    deg = hist_ref[0, :, 0:1] + hist_ref[1, :, 0:1] + 1.0
    rows = i * BLK + lax.broadcasted_iota(jnp.int32, (BLK, 1), 0)
    dinv = jnp.where(rows < N, 1.0 / jnp.sqrt(deg), 0.0)
    dinv32 = jnp.broadcast_to(dinv, (BLK, D1))
    y1_ref[...] = dinv32 * u
    dinv_ref[...] = dinv32


def _k2(X, W1, hist):
    return pl.pallas_call(
        _k2_body,
        grid=(ROWS // BLK,),
        in_specs=[
            pl.BlockSpec((BLK, 128), lambda i: (i, 0)),
            pl.BlockSpec((128, D1), lambda i: (0, 0)),
            pl.BlockSpec((NC, BLK, DH), lambda i: (0, i, 0)),
        ],
        out_specs=[
            pl.BlockSpec((BLK, D1), lambda i: (i, 0)),
            pl.BlockSpec((BLK, D1), lambda i: (i, 0)),
        ],
        out_shape=[
            jax.ShapeDtypeStruct((ROWS, D1), jnp.float32),
            jax.ShapeDtypeStruct((ROWS, D1), jnp.float32),
        ],
    )(X, W1, hist)


def _k3_body(s1_ref, y1_ref, dinv_ref, b1_ref, y2_ref):
    dinv = dinv_ref[...]
    agg = dinv * (s1_ref[0] + s1_ref[1] + y1_ref[...]) + b1_ref[...]
    y2_ref[...] = dinv * jnp.maximum(agg, 0.0)


def _k3(s1, y1, dinv32, b1):
    return pl.pallas_call(
        _k3_body,
        grid=(ROWS // BLK,),
        in_specs=[
            pl.BlockSpec((NC, BLK, D1), lambda i: (0, i, 0)),
            pl.BlockSpec((BLK, D1), lambda i: (i, 0)),
            pl.BlockSpec((BLK, D1), lambda i: (i, 0)),
            pl.BlockSpec((1, D1), lambda i: (0, 0)),
        ],
        out_specs=pl.BlockSpec((BLK, D1), lambda i: (i, 0)),
        out_shape=jax.ShapeDtypeStruct((ROWS, D1), jnp.float32),
    )(s1, y1, dinv32, b1)


def _k4_body(s2_ref, y2_ref, dinv_ref, wmu_ref, bmu_ref, wsig_ref, bsig_ref,
             eps_ref, mus_ref, lsig_ref, z_ref):
    i = pl.program_id(0)
    agg = dinv_ref[...] * (s2_ref[0] + s2_ref[1] + y2_ref[...])
    mus = jnp.dot(agg, wmu_ref[...], preferred_element_type=jnp.float32,
                  precision=lax.Precision.HIGHEST) + bmu_ref[...]
    lsig = jnp.dot(agg, wsig_ref[...], preferred_element_type=jnp.float32,
                   precision=lax.Precision.HIGHEST) + bsig_ref[...]
    z = mus + jnp.exp(0.5 * lsig) * eps_ref[...]
    rows = i * BLK + lax.broadcasted_iota(jnp.int32, (BLK, 1), 0)
    mus_ref[...] = mus
    lsig_ref[...] = lsig
    z_ref[...] = jnp.where(rows < N, z, 0.0)


def _k4(s2, y2, dinv32, Wmu, bmu, Wsig, bsig, eps):
    return pl.pallas_call(
        _k4_body,
        grid=(ROWS // BLK,),
        in_specs=[
            pl.BlockSpec((NC, BLK, D1), lambda i: (0, i, 0)),
            pl.BlockSpec((BLK, D1), lambda i: (i, 0)),
            pl.BlockSpec((BLK, D1), lambda i: (i, 0)),
            pl.BlockSpec((D1, DZ), lambda i: (0, 0)),
            pl.BlockSpec((1, DZ), lambda i: (0, 0)),
            pl.BlockSpec((D1, DZ), lambda i: (0, 0)),
            pl.BlockSpec((1, DZ), lambda i: (0, 0)),
            pl.BlockSpec((BLK, DZ), lambda i: (i, 0)),
        ],
        out_specs=[
            pl.BlockSpec((BLK, DZ), lambda i: (i, 0)),
            pl.BlockSpec((BLK, DZ), lambda i: (i, 0)),
            pl.BlockSpec((BLK, DZ), lambda i: (i, 0)),
        ],
        out_shape=[
            jax.ShapeDtypeStruct((N, DZ), jnp.float32),
            jax.ShapeDtypeStruct((N, DZ), jnp.float32),
            jax.ShapeDtypeStruct((ROWS, DZ), jnp.float32),
        ],
    )(s2, y2, dinv32, Wmu, bmu, Wsig, bsig, eps)


BM, BN = 2560, 2560


def _k5_body(zl_ref, zr_ref, out_ref):
    out_ref[...] = lax.dot_general(
        zl_ref[...], zr_ref[...], (((1,), (1,)), ((), ())),
        preferred_element_type=jnp.float32,
        precision=lax.Precision.DEFAULT)


def _k5(z):
    return pl.pallas_call(
        _k5_body,
        grid=(pl.cdiv(N, BM), pl.cdiv(N, BN)),
        in_specs=[
            pl.BlockSpec((BM, DZ), lambda i, j: (i, 0)),
            pl.BlockSpec((BN, DZ), lambda i, j: (j, 0)),
        ],
        out_specs=pl.BlockSpec((BM, BN), lambda i, j: (i, j)),
        out_shape=jax.ShapeDtypeStruct((N, N), jnp.float32),
    )(z, z)


def kernel(X, graph, W1, b1, Wmu, bmu, Wsig, bsig, eps):
    g = graph.reshape(2, NW, CH, CB)
    zrow16 = jnp.zeros((RPT, DH), jnp.float32)
    zrow32 = jnp.zeros((RPT, D1), jnp.float32)
    onerow = jnp.concatenate(
        [jnp.ones((CB, 1), jnp.float32), jnp.zeros((CB, DH - 1), jnp.float32)],
        axis=1)

    hist = _sc_hist(g, onerow, zrow16)
    y1, dinv32 = _k2(X, W1, hist)
    s1 = _sc_agg(g, y1, zrow32)
    y2 = _k3(s1, y1, dinv32, b1.reshape(1, D1))
    s2 = _sc_agg(g, y2, zrow32)
    mus, lsig, z = _k4(s2, y2, dinv32, Wmu, bmu.reshape(1, DZ),
                       Wsig, bsig.reshape(1, DZ), eps)
    zzt = _k5(z)
    return (zzt, mus, lsig)
